# Initial kernel scaffold; baseline (speedup 1.0000x reference)
import jax
import jax.numpy as jnp
from jax.experimental import pallas as pl


def kernel(vector_embeddings, scalar_embeddings, neighbour_index, neighbour_vectors, Wf, bf, W1, b1, W2, b2):
    raise NotImplementedError("write your pallas kernel here")


# speedup vs baseline: 27.9129x; 27.9129x over previous
import functools

import jax
import jax.numpy as jnp
from jax import lax
from jax.experimental import pallas as pl
from jax.experimental.pallas import tpu as pltpu
from jax.experimental.pallas import tpu_sc as plsc

CUT = 5.0
NATOMS = 10000
NEDGES = 320000
DIM = 128
FW = 128
NW = 32
PW = 10240
EPAD = NW * PW
NCH = PW // 128
NAPAD = 10240
NPT = NAPAD // 16

BE = 4096
BN = 2000


def _edge_body(nv_ref, out_ref):
    nv = nv_ref[...]
    x, y, z = nv[:, 0:1], nv[:, 1:2], nv[:, 2:3]
    d = jnp.sqrt(x * x + y * y + z * z)
    inv = 1.0 / d
    u = nv * inv
    t = d / CUT
    t2 = t * t
    t3 = t2 * t
    t6 = t3 * t3
    t7 = t6 * t
    t8 = t7 * t
    env = jnp.where(d < CUT, 1.0 - 28.0 * t6 + 48.0 * t7 - 21.0 * t8, 0.0)
    n = (lax.broadcasted_iota(jnp.int32, (1, 8), 1) + 1).astype(jnp.float32)
    s = jnp.sin(d * (jnp.pi / CUT) * n)
    ge = (jnp.sqrt(2.0 / CUT) * inv * env) * s
    g12 = jnp.concatenate([ge, env, jnp.zeros_like(nv)], axis=1)
    pad = jnp.zeros((g12.shape[0], FW - 48), jnp.float32)
    out_ref[...] = jnp.concatenate(
        [g12, g12 * u[:, 0:1], g12 * u[:, 1:2], g12 * u[:, 2:3], pad], axis=1)


def _edge_feats(nv_pad):
    return pl.pallas_call(
        _edge_body,
        grid=(EPAD // BE,),
        in_specs=[pl.BlockSpec((BE, 3), lambda i: (i, 0))],
        out_specs=pl.BlockSpec((BE, FW), lambda i: (i, 0)),
        out_shape=jax.ShapeDtypeStruct((EPAD, FW), jnp.float32),
    )(nv_pad)


def _sc_scatter_body(feats_hbm, idx_hbm, zeros_hbm, out_hbm, idx_c, fbuf, acc):
    c = lax.axis_index("c")
    s = lax.axis_index("s")
    wid = s * 2 + c
    def zrow(t, carry):
        pltpu.sync_copy(zeros_hbm.at[pl.ds(s * NPT + t * 128, 128)], fbuf)
        pltpu.sync_copy(fbuf, acc.at[pl.ds(s * NPT + t * 128, 128)])
        return carry

    lax.fori_loop(0, NPT // 128, zrow, 0)
    plsc.subcore_barrier()
    base = wid * PW

    def chunk(j, carry):
        pltpu.sync_copy(idx_hbm.at[wid].at[j], idx_c)
        pltpu.sync_copy(feats_hbm.at[pl.ds(base + j * 128, 128)], fbuf)
        pltpu.sync_copy(fbuf, acc.at[idx_c], add=True)
        return carry

    lax.fori_loop(0, NCH, chunk, 0)
    plsc.subcore_barrier()

    def orow(t, carry):
        pltpu.sync_copy(acc.at[pl.ds(s * NPT + t * 128, 128)], fbuf)
        pltpu.sync_copy(fbuf, out_hbm.at[c].at[pl.ds(s * NPT + t * 128, 128)])
        return carry

    lax.fori_loop(0, NPT // 128, orow, 0)


def _sc_scatter(feats, idx3, zeros):
    mesh = plsc.VectorSubcoreMesh(core_axis_name="c", subcore_axis_name="s")
    call = functools.partial(
        pl.kernel,
        mesh=mesh,
        out_type=jax.ShapeDtypeStruct((2, NAPAD, FW), jnp.float32),
        scratch_types=[
            pltpu.VMEM((128,), jnp.int32),
            pltpu.VMEM((128, FW), jnp.float32),
            pltpu.VMEM_SHARED((NAPAD, FW), jnp.float32),
        ],
    )(_sc_scatter_body)
    return call(feats, idx3, zeros)


def _dense_body(m_ref, s_ref, v_ref, w1_ref, b1_ref, w2_ref, b2_ref,
                wf_ref, wb_ref, rsel_ref, ds_ref, dv_ref):
    m = m_ref[0] + m_ref[1]
    h = jnp.dot(s_ref[...], w1_ref[...],
                preferred_element_type=jnp.float32) + b1_ref[...]
    h = h * jax.nn.sigmoid(h)
    phi = jnp.dot(h, w2_ref[...],
                  preferred_element_type=jnp.float32) + b2_ref[...]
    f = jnp.dot(m, wf_ref[...], preferred_element_type=jnp.float32)
    b384 = jnp.dot(m, wb_ref[...], preferred_element_type=jnp.float32)
    ds_ref[...] = phi[:, :DIM] * f[:, :DIM]
    pb = phi[:, DIM:2 * DIM]
    pc = phi[:, 2 * DIM:] * f[:, 2 * DIM:]
    rsel = rsel_ref[...]
    pb_rep = jnp.dot(pb, rsel, preferred_element_type=jnp.float32)
    pc_rep = jnp.dot(pc, rsel, preferred_element_type=jnp.float32)
    dv_ref[...] = pb_rep * b384 + pc_rep * v_ref[...]


def _dense(msc, s_emb, v384, w1, b1, w2, b2, wf48, wb48, rsel):
    return pl.pallas_call(
        _dense_body,
        grid=(NATOMS // BN,),
        in_specs=[
            pl.BlockSpec((2, BN, FW), lambda i: (0, i, 0)),
            pl.BlockSpec((BN, DIM), lambda i: (i, 0)),
            pl.BlockSpec((BN, 3 * DIM), lambda i: (i, 0)),
            pl.BlockSpec((DIM, DIM), lambda i: (0, 0)),
            pl.BlockSpec((1, DIM), lambda i: (0, 0)),
            pl.BlockSpec((DIM, 3 * DIM), lambda i: (0, 0)),
            pl.BlockSpec((1, 3 * DIM), lambda i: (0, 0)),
            pl.BlockSpec((FW, 3 * DIM), lambda i: (0, 0)),
            pl.BlockSpec((FW, 3 * DIM), lambda i: (0, 0)),
            pl.BlockSpec((DIM, 3 * DIM), lambda i: (0, 0)),
        ],
        out_specs=[
            pl.BlockSpec((BN, DIM), lambda i: (i, 0)),
            pl.BlockSpec((BN, 3 * DIM), lambda i: (i, 0)),
        ],
        out_shape=[
            jax.ShapeDtypeStruct((NATOMS, DIM), jnp.float32),
            jax.ShapeDtypeStruct((NATOMS, 3 * DIM), jnp.float32),
        ],
    )(msc, s_emb, v384, w1, b1, w2, b2, wf48, wb48, rsel)


def kernel(vector_embeddings, scalar_embeddings, neighbour_index,
           neighbour_vectors, Wf, bf, W1, b1, W2, b2):
    f32 = jnp.float32
    npad = EPAD - NEDGES
    nv_pad = jnp.concatenate(
        [neighbour_vectors,
         jnp.concatenate([jnp.full((npad, 1), 2.0 * CUT, f32),
                          jnp.zeros((npad, 2), f32)], axis=1)], axis=0)
    idx3 = jnp.concatenate(
        [neighbour_index[1], jnp.zeros((npad,), jnp.int32)]).reshape(NW, NCH, 128)

    feats = _edge_feats(nv_pad)
    msc = _sc_scatter(feats, idx3, jnp.zeros((NAPAD, FW), f32))

    wfe = jnp.concatenate([Wf, bf[None, :]], axis=0)
    wf48 = jnp.concatenate([wfe, jnp.zeros((FW - 9, 3 * DIM), f32)], axis=0)
    wfb = wfe[:, DIM:2 * DIM]
    blocks = [jnp.zeros((12, 3 * DIM), f32)]
    for k in range(3):
        cols = [wfb if kk == k else jnp.zeros_like(wfb) for kk in range(3)]
        blk = jnp.stack(cols, axis=-1).reshape(9, 3 * DIM)
        blocks.append(jnp.concatenate([blk, jnp.zeros((3, 3 * DIM), f32)], 0))
    blocks.append(jnp.zeros((FW - 48, 3 * DIM), f32))
    wb48 = jnp.concatenate(blocks, axis=0)
    rsel = jnp.broadcast_to(jnp.eye(DIM, dtype=f32)[:, :, None],
                            (DIM, DIM, 3)).reshape(DIM, 3 * DIM)

    v384 = vector_embeddings.reshape(NATOMS, 3 * DIM)
    ds, dv384 = _dense(msc, scalar_embeddings, v384,
                       W1, b1[None, :], W2, b2[None, :], wf48, wb48, rsel)
    return (dv384.reshape(NATOMS, DIM, 3), ds)


# speedup vs baseline: 87.4134x; 3.1317x over previous
import functools

import jax
import jax.numpy as jnp
from jax import lax
from jax.experimental import pallas as pl
from jax.experimental.pallas import tpu as pltpu
from jax.experimental.pallas import tpu_sc as plsc

CUT = 5.0
NATOMS = 10000
NEDGES = 320000
DIM = 128
FW = 128
NW = 32
PW = 10240
EPAD = NW * PW
NCH = PW // 128
NAPAD = 10240
NPT = NAPAD // 16

BE = 4096
BN = 2000


BR = 32


def _edge_body(xs_ref, ys_ref, zs_ref, out_ref):
    x, y, z = xs_ref[...], ys_ref[...], zs_ref[...]
    d = jnp.sqrt(x * x + y * y + z * z)
    inv = 1.0 / d
    ux, uy, uz = x * inv, y * inv, z * inv
    t = d / CUT
    t2 = t * t
    t3 = t2 * t
    t6 = t3 * t3
    t7 = t6 * t
    t8 = t7 * t
    env = jnp.where(d < CUT, 1.0 - 28.0 * t6 + 48.0 * t7 - 21.0 * t8, 0.0)
    coef = jnp.sqrt(2.0 / CUT) * inv * env
    zro = jnp.zeros_like(x)
    g12 = [coef * jnp.sin(d * (float(n) * jnp.pi / CUT)) for n in range(1, 9)]
    g12 += [env, zro, zro, zro]
    planes = g12 + [p * ux for p in g12] + [p * uy for p in g12] \
        + [p * uz for p in g12]
    p48 = jnp.stack(planes, axis=1)
    rows = jnp.transpose(p48, (0, 2, 1))
    pad = jnp.zeros((BR, 128, FW - 48), jnp.float32)
    out_ref[...] = jnp.concatenate([rows, pad], axis=2)


def _edge_feats(xs, ys, zs):
    out3 = pl.pallas_call(
        _edge_body,
        grid=(EPAD // (BR * 128),),
        in_specs=[pl.BlockSpec((BR, 128), lambda i: (i, 0))] * 3,
        out_specs=pl.BlockSpec((BR, 128, FW), lambda i: (i, 0, 0)),
        out_shape=jax.ShapeDtypeStruct((EPAD // 128, 128, FW), jnp.float32),
    )(xs, ys, zs)
    return out3.reshape(EPAD, FW)


def _sc_scatter_body(feats_hbm, idx_hbm, zeros_hbm, out_hbm, idx_c, fbuf, acc):
    c = lax.axis_index("c")
    s = lax.axis_index("s")
    wid = s * 2 + c
    def zrow(t, carry):
        pltpu.sync_copy(zeros_hbm.at[pl.ds(s * NPT + t * 128, 128)], fbuf)
        pltpu.sync_copy(fbuf, acc.at[pl.ds(s * NPT + t * 128, 128)])
        return carry

    lax.fori_loop(0, NPT // 128, zrow, 0)
    plsc.subcore_barrier()
    base = wid * PW

    def chunk(j, carry):
        pltpu.sync_copy(idx_hbm.at[wid].at[j], idx_c)
        pltpu.sync_copy(feats_hbm.at[pl.ds(base + j * 128, 128)], fbuf)
        pltpu.sync_copy(fbuf, acc.at[idx_c], add=True)
        return carry

    lax.fori_loop(0, NCH, chunk, 0)
    plsc.subcore_barrier()

    def orow(t, carry):
        pltpu.sync_copy(acc.at[pl.ds(s * NPT + t * 128, 128)], fbuf)
        pltpu.sync_copy(fbuf, out_hbm.at[c].at[pl.ds(s * NPT + t * 128, 128)])
        return carry

    lax.fori_loop(0, NPT // 128, orow, 0)


def _sc_scatter(feats, idx3, zeros):
    mesh = plsc.VectorSubcoreMesh(core_axis_name="c", subcore_axis_name="s")
    call = functools.partial(
        pl.kernel,
        mesh=mesh,
        out_type=jax.ShapeDtypeStruct((2, NAPAD, FW), jnp.float32),
        scratch_types=[
            pltpu.VMEM((128,), jnp.int32),
            pltpu.VMEM((128, FW), jnp.float32),
            pltpu.VMEM_SHARED((NAPAD, FW), jnp.float32),
        ],
    )(_sc_scatter_body)
    return call(feats, idx3, zeros)


def _dense_body(m_ref, s_ref, v_ref, w1_ref, b1_ref, w2_ref, b2_ref,
                wf_ref, wb_ref, rsel_ref, ds_ref, dv_ref):
    m = m_ref[0] + m_ref[1]
    h = jnp.dot(s_ref[...], w1_ref[...],
                preferred_element_type=jnp.float32) + b1_ref[...]
    h = h * jax.nn.sigmoid(h)
    phi = jnp.dot(h, w2_ref[...],
                  preferred_element_type=jnp.float32) + b2_ref[...]
    f = jnp.dot(m, wf_ref[...], preferred_element_type=jnp.float32)
    b384 = jnp.dot(m, wb_ref[...], preferred_element_type=jnp.float32)
    ds_ref[...] = phi[:, :DIM] * f[:, :DIM]
    pb = phi[:, DIM:2 * DIM]
    pc = phi[:, 2 * DIM:] * f[:, 2 * DIM:]
    rsel = rsel_ref[...]
    pb_rep = jnp.dot(pb, rsel, preferred_element_type=jnp.float32)
    pc_rep = jnp.dot(pc, rsel, preferred_element_type=jnp.float32)
    dv_ref[...] = pb_rep * b384 + pc_rep * v_ref[...]


def _dense(msc, s_emb, v384, w1, b1, w2, b2, wf48, wb48, rsel):
    return pl.pallas_call(
        _dense_body,
        grid=(NATOMS // BN,),
        in_specs=[
            pl.BlockSpec((2, BN, FW), lambda i: (0, i, 0)),
            pl.BlockSpec((BN, DIM), lambda i: (i, 0)),
            pl.BlockSpec((BN, 3 * DIM), lambda i: (i, 0)),
            pl.BlockSpec((DIM, DIM), lambda i: (0, 0)),
            pl.BlockSpec((1, DIM), lambda i: (0, 0)),
            pl.BlockSpec((DIM, 3 * DIM), lambda i: (0, 0)),
            pl.BlockSpec((1, 3 * DIM), lambda i: (0, 0)),
            pl.BlockSpec((FW, 3 * DIM), lambda i: (0, 0)),
            pl.BlockSpec((FW, 3 * DIM), lambda i: (0, 0)),
            pl.BlockSpec((DIM, 3 * DIM), lambda i: (0, 0)),
        ],
        out_specs=[
            pl.BlockSpec((BN, DIM), lambda i: (i, 0)),
            pl.BlockSpec((BN, 3 * DIM), lambda i: (i, 0)),
        ],
        out_shape=[
            jax.ShapeDtypeStruct((NATOMS, DIM), jnp.float32),
            jax.ShapeDtypeStruct((NATOMS, 3 * DIM), jnp.float32),
        ],
    )(msc, s_emb, v384, w1, b1, w2, b2, wf48, wb48, rsel)


def kernel(vector_embeddings, scalar_embeddings, neighbour_index,
           neighbour_vectors, Wf, bf, W1, b1, W2, b2):
    f32 = jnp.float32
    npad = EPAD - NEDGES
    nv_pad = jnp.concatenate(
        [neighbour_vectors,
         jnp.concatenate([jnp.full((npad, 1), 2.0 * CUT, f32),
                          jnp.zeros((npad, 2), f32)], axis=1)], axis=0)
    idx3 = jnp.concatenate(
        [neighbour_index[1], jnp.zeros((npad,), jnp.int32)]).reshape(NW, NCH, 128)

    xs = nv_pad[:, 0].reshape(EPAD // 128, 128)
    ys = nv_pad[:, 1].reshape(EPAD // 128, 128)
    zs = nv_pad[:, 2].reshape(EPAD // 128, 128)
    feats = _edge_feats(xs, ys, zs)
    msc = _sc_scatter(feats, idx3, jnp.zeros((NAPAD, FW), f32))

    wfe = jnp.concatenate([Wf, bf[None, :]], axis=0)
    wf48 = jnp.concatenate([wfe, jnp.zeros((FW - 9, 3 * DIM), f32)], axis=0)
    wfb = wfe[:, DIM:2 * DIM]
    blocks = [jnp.zeros((12, 3 * DIM), f32)]
    for k in range(3):
        cols = [wfb if kk == k else jnp.zeros_like(wfb) for kk in range(3)]
        blk = jnp.stack(cols, axis=-1).reshape(9, 3 * DIM)
        blocks.append(jnp.concatenate([blk, jnp.zeros((3, 3 * DIM), f32)], 0))
    blocks.append(jnp.zeros((FW - 48, 3 * DIM), f32))
    wb48 = jnp.concatenate(blocks, axis=0)
    rsel = jnp.broadcast_to(jnp.eye(DIM, dtype=f32)[:, :, None],
                            (DIM, DIM, 3)).reshape(DIM, 3 * DIM)

    v384 = vector_embeddings.reshape(NATOMS, 3 * DIM)
    ds, dv384 = _dense(msc, scalar_embeddings, v384,
                       W1, b1[None, :], W2, b2[None, :], wf48, wb48, rsel)
    return (dv384.reshape(NATOMS, DIM, 3), ds)


# speedup vs baseline: 101.2476x; 1.1583x over previous
import functools

import jax
import jax.numpy as jnp
from jax import lax
from jax.experimental import pallas as pl
from jax.experimental.pallas import tpu as pltpu
from jax.experimental.pallas import tpu_sc as plsc

CUT = 5.0
NATOMS = 10000
NEDGES = 320000
DIM = 128
FW = 128
NW = 32
PW = 10240
EPAD = NW * PW
NCH = PW // 128
NAPAD = 10240
NPT = NAPAD // 16

BE = 4096
BN = 2000


BR = 32


def _edge_body(xs_ref, ys_ref, zs_ref, out_ref):
    x, y, z = xs_ref[...], ys_ref[...], zs_ref[...]
    d = jnp.sqrt(x * x + y * y + z * z)
    inv = 1.0 / d
    ux, uy, uz = x * inv, y * inv, z * inv
    t = d / CUT
    t2 = t * t
    t3 = t2 * t
    t6 = t3 * t3
    t7 = t6 * t
    t8 = t7 * t
    env = jnp.where(d < CUT, 1.0 - 28.0 * t6 + 48.0 * t7 - 21.0 * t8, 0.0)
    coef = jnp.sqrt(2.0 / CUT) * inv * env
    zro = jnp.zeros_like(x)
    g12 = [coef * jnp.sin(d * (float(n) * jnp.pi / CUT)) for n in range(1, 9)]
    g12 += [env, zro, zro, zro]
    planes = g12 + [p * ux for p in g12] + [p * uy for p in g12] \
        + [p * uz for p in g12]
    p48 = jnp.stack(planes, axis=1)
    rows = jnp.transpose(p48, (0, 2, 1))
    pad = jnp.zeros((BR, 128, FW - 48), jnp.float32)
    out_ref[...] = jnp.concatenate([rows, pad], axis=2)


def _edge_feats(xs, ys, zs):
    out3 = pl.pallas_call(
        _edge_body,
        grid=(EPAD // (BR * 128),),
        in_specs=[pl.BlockSpec((BR, 128), lambda i: (i, 0))] * 3,
        out_specs=pl.BlockSpec((BR, 128, FW), lambda i: (i, 0, 0)),
        out_shape=jax.ShapeDtypeStruct((EPAD // 128, 128, FW), jnp.float32),
    )(xs, ys, zs)
    return out3.reshape(EPAD, FW)


OVR = 128 * 128 // FW


def _sc_scatter_body(feats_hbm, idx_hbm, zeros_hbm, out_hbm,
                     idx_v, fb0, fb1, sem0, sem1, acc):
    c = lax.axis_index("c")
    s = lax.axis_index("s")
    wid = s * 2 + c
    sbuf = fb0.at[pl.ds(0, 128)]
    def zrow(t, carry):
        pltpu.sync_copy(zeros_hbm.at[pl.ds(s * NPT + t * 128, 128)], sbuf)
        pltpu.sync_copy(sbuf, acc.at[pl.ds(s * NPT + t * 128, 128)])
        return carry

    lax.fori_loop(0, NPT // 128, zrow, 0)
    plsc.subcore_barrier()
    base = wid * PW
    pltpu.sync_copy(idx_hbm.at[wid], idx_v)

    def pair(i, carry):
        j0 = i * 2
        h0 = pltpu.async_copy(feats_hbm.at[pl.ds(base + j0 * 128, 128)],
                              fb0.at[pl.ds(0, 128)], sem0)
        h1 = pltpu.async_copy(feats_hbm.at[pl.ds(base + (j0 + 1) * 128, 128)],
                              fb1.at[pl.ds(0, 128)], sem1)
        h0.wait()
        pltpu.sync_copy(fb0, acc.at[idx_v.at[j0]], add=True)
        h1.wait()
        pltpu.sync_copy(fb1, acc.at[idx_v.at[j0 + 1]], add=True)
        return carry

    lax.fori_loop(0, NCH // 2, pair, 0)
    plsc.subcore_barrier()

    def orow(t, carry):
        pltpu.sync_copy(acc.at[pl.ds(s * NPT + t * 128, 128)], sbuf)
        pltpu.sync_copy(sbuf, out_hbm.at[c].at[pl.ds(s * NPT + t * 128, 128)])
        return carry

    lax.fori_loop(0, NPT // 128, orow, 0)


def _sc_scatter(feats, idx3, zeros):
    mesh = plsc.VectorSubcoreMesh(core_axis_name="c", subcore_axis_name="s")
    call = functools.partial(
        pl.kernel,
        mesh=mesh,
        out_type=jax.ShapeDtypeStruct((2, NAPAD, FW), jnp.float32),
        scratch_types=[
            pltpu.VMEM((NCH, 128), jnp.int32),
            pltpu.VMEM((OVR, FW), jnp.float32),
            pltpu.VMEM((OVR, FW), jnp.float32),
            pltpu.SemaphoreType.DMA,
            pltpu.SemaphoreType.DMA,
            pltpu.VMEM_SHARED((NAPAD, FW), jnp.float32),
        ],
    )(_sc_scatter_body)
    return call(feats, idx3, zeros)


def _dense_body(m_ref, s_ref, v_ref, w1_ref, b1_ref, w2_ref, b2_ref,
                wf_ref, wb_ref, rsel_ref, ds_ref, dv_ref):
    m = m_ref[0] + m_ref[1]
    h = jnp.dot(s_ref[...], w1_ref[...],
                preferred_element_type=jnp.float32) + b1_ref[...]
    h = h * jax.nn.sigmoid(h)
    phi = jnp.dot(h, w2_ref[...],
                  preferred_element_type=jnp.float32) + b2_ref[...]
    f = jnp.dot(m, wf_ref[...], preferred_element_type=jnp.float32)
    b384 = jnp.dot(m, wb_ref[...], preferred_element_type=jnp.float32)
    ds_ref[...] = phi[:, :DIM] * f[:, :DIM]
    pb = phi[:, DIM:2 * DIM]
    pc = phi[:, 2 * DIM:] * f[:, 2 * DIM:]
    rsel = rsel_ref[...]
    pb_rep = jnp.dot(pb, rsel, preferred_element_type=jnp.float32)
    pc_rep = jnp.dot(pc, rsel, preferred_element_type=jnp.float32)
    dv_ref[...] = pb_rep * b384 + pc_rep * v_ref[...]


def _dense(msc, s_emb, v384, w1, b1, w2, b2, wf48, wb48, rsel):
    return pl.pallas_call(
        _dense_body,
        grid=(NATOMS // BN,),
        in_specs=[
            pl.BlockSpec((2, BN, FW), lambda i: (0, i, 0)),
            pl.BlockSpec((BN, DIM), lambda i: (i, 0)),
            pl.BlockSpec((BN, 3 * DIM), lambda i: (i, 0)),
            pl.BlockSpec((DIM, DIM), lambda i: (0, 0)),
            pl.BlockSpec((1, DIM), lambda i: (0, 0)),
            pl.BlockSpec((DIM, 3 * DIM), lambda i: (0, 0)),
            pl.BlockSpec((1, 3 * DIM), lambda i: (0, 0)),
            pl.BlockSpec((FW, 3 * DIM), lambda i: (0, 0)),
            pl.BlockSpec((FW, 3 * DIM), lambda i: (0, 0)),
            pl.BlockSpec((DIM, 3 * DIM), lambda i: (0, 0)),
        ],
        out_specs=[
            pl.BlockSpec((BN, DIM), lambda i: (i, 0)),
            pl.BlockSpec((BN, 3 * DIM), lambda i: (i, 0)),
        ],
        out_shape=[
            jax.ShapeDtypeStruct((NATOMS, DIM), jnp.float32),
            jax.ShapeDtypeStruct((NATOMS, 3 * DIM), jnp.float32),
        ],
    )(msc, s_emb, v384, w1, b1, w2, b2, wf48, wb48, rsel)


def kernel(vector_embeddings, scalar_embeddings, neighbour_index,
           neighbour_vectors, Wf, bf, W1, b1, W2, b2):
    f32 = jnp.float32
    npad = EPAD - NEDGES
    nv_pad = jnp.concatenate(
        [neighbour_vectors,
         jnp.concatenate([jnp.full((npad, 1), 2.0 * CUT, f32),
                          jnp.zeros((npad, 2), f32)], axis=1)], axis=0)
    idx3 = jnp.concatenate(
        [neighbour_index[1], jnp.zeros((npad,), jnp.int32)]).reshape(NW, NCH, 128)

    xs = nv_pad[:, 0].reshape(EPAD // 128, 128)
    ys = nv_pad[:, 1].reshape(EPAD // 128, 128)
    zs = nv_pad[:, 2].reshape(EPAD // 128, 128)
    feats = _edge_feats(xs, ys, zs)
    msc = _sc_scatter(feats, idx3, jnp.zeros((NAPAD, FW), f32))

    wfe = jnp.concatenate([Wf, bf[None, :]], axis=0)
    wf48 = jnp.concatenate([wfe, jnp.zeros((FW - 9, 3 * DIM), f32)], axis=0)
    wfb = wfe[:, DIM:2 * DIM]
    blocks = [jnp.zeros((12, 3 * DIM), f32)]
    for k in range(3):
        cols = [wfb if kk == k else jnp.zeros_like(wfb) for kk in range(3)]
        blk = jnp.stack(cols, axis=-1).reshape(9, 3 * DIM)
        blocks.append(jnp.concatenate([blk, jnp.zeros((3, 3 * DIM), f32)], 0))
    blocks.append(jnp.zeros((FW - 48, 3 * DIM), f32))
    wb48 = jnp.concatenate(blocks, axis=0)
    rsel = jnp.broadcast_to(jnp.eye(DIM, dtype=f32)[:, :, None],
                            (DIM, DIM, 3)).reshape(DIM, 3 * DIM)

    v384 = vector_embeddings.reshape(NATOMS, 3 * DIM)
    ds, dv384 = _dense(msc, scalar_embeddings, v384,
                       W1, b1[None, :], W2, b2[None, :], wf48, wb48, rsel)
    return (dv384.reshape(NATOMS, DIM, 3), ds)
